# Initial kernel scaffold; baseline (speedup 1.0000x reference)
import jax
import jax.numpy as jnp
from jax.experimental import pallas as pl


def kernel(elem_weights, elem_fea, self_fea_idx, nbr_fea_idx, cry_elem_idx, params):
    raise NotImplementedError("write your pallas kernel here")


# speedup vs baseline: 2.5252x; 2.5252x over previous
import functools

import jax
import jax.numpy as jnp
import numpy as np
from jax.experimental import pallas as pl
from jax.experimental.pallas import tpu as pltpu

N_NODES = 10000
N_EDGES = 320000
N_CRYSTALS = 2000
F = 64
H3 = 192
NBLK = 1000
EBLK = 3200

_INTERPRET = False


def _pcall(body, out_shape, in_specs, out_specs, grid):
    return pl.pallas_call(
        body,
        out_shape=out_shape,
        in_specs=in_specs,
        out_specs=out_specs,
        grid=grid,
        interpret=_INTERPRET,
    )


def _embed(elem_fea, w, We64, be64, lastcol):
    def body(ef_ref, w_ref, We_ref, be_ref, lc_ref, x_ref):
        x = jnp.dot(ef_ref[...], We_ref[...], preferred_element_type=jnp.float32)
        x = x + be_ref[...]
        x_ref[...] = x + jnp.dot(w_ref[...], lc_ref[...],
                                 preferred_element_type=jnp.float32)

    return _pcall(
        body,
        jax.ShapeDtypeStruct((N_NODES, F), jnp.float32),
        [
            pl.BlockSpec((NBLK, 128), lambda i: (i, 0)),
            pl.BlockSpec((NBLK, 1), lambda i: (i, 0)),
            pl.BlockSpec((128, F), lambda i: (0, 0)),
            pl.BlockSpec((1, F), lambda i: (0, 0)),
            pl.BlockSpec((1, F), lambda i: (0, 0)),
        ],
        pl.BlockSpec((NBLK, F), lambda i: (i, 0)),
        (N_NODES // NBLK,),
    )(elem_fea, w, We64, be64, lastcol)


def _edge_fwd(xs, xn, lwn, W1s, W1n, b1, Wg2, bg2):
    def body(xs_ref, xn_ref, lwn_ref, W1s_ref, W1n_ref, b1_ref, Wg2_ref,
             bg2_ref, g8_ref, h_ref):
        z = jnp.dot(xs_ref[...], W1s_ref[...], preferred_element_type=jnp.float32)
        z = z + jnp.dot(xn_ref[...], W1n_ref[...], preferred_element_type=jnp.float32)
        z = z + b1_ref[...]
        hid = jnp.where(z > 0, z, 0.01 * z)
        g = jnp.dot(hid, Wg2_ref[...], preferred_element_type=jnp.float32)
        g = g + bg2_ref[...]
        g8_ref[...] = jnp.concatenate(
            [g, lwn_ref[...], jnp.zeros((EBLK, 2), jnp.float32)], axis=1)
        h_ref[...] = hid[:, H3:]

    return _pcall(
        body,
        [jax.ShapeDtypeStruct((N_EDGES, 8), jnp.float32),
         jax.ShapeDtypeStruct((N_EDGES, H3), jnp.float32)],
        [
            pl.BlockSpec((EBLK, F), lambda i: (i, 0)),
            pl.BlockSpec((EBLK, F), lambda i: (i, 0)),
            pl.BlockSpec((EBLK, 3), lambda i: (i, 0)),
            pl.BlockSpec((F, 2 * H3), lambda i: (0, 0)),
            pl.BlockSpec((F, 2 * H3), lambda i: (0, 0)),
            pl.BlockSpec((1, 2 * H3), lambda i: (0, 0)),
            pl.BlockSpec((2 * H3, 3), lambda i: (0, 0)),
            pl.BlockSpec((1, 3), lambda i: (0, 0)),
        ],
        [pl.BlockSpec((EBLK, 8), lambda i: (i, 0)),
         pl.BlockSpec((EBLK, H3), lambda i: (i, 0))],
        (N_EDGES // EBLK,),
    )(xs, xn, lwn, W1s, W1n, b1, Wg2, bg2)


def _node_fin(V, D, x, Wcat, Bstk, nrows, nblk, residual):
    def body2(V_ref, D_ref, x_ref, Wc_ref, Bs_ref, o_ref):
        D = D_ref[...]
        sig = D / (D + 1e-10)
        Vn = V_ref[...] * jnp.repeat(1.0 / (D + 1e-10), F, axis=1)
        o = jnp.dot(Vn, Wc_ref[...], preferred_element_type=jnp.float32)
        o = o + jnp.dot(sig, Bs_ref[...], preferred_element_type=jnp.float32)
        if residual:
            o = o + x_ref[...]
        o_ref[...] = o

    return _pcall(
        body2,
        jax.ShapeDtypeStruct((nrows, F), jnp.float32),
        [
            pl.BlockSpec((nblk, H3), lambda i: (i, 0)),
            pl.BlockSpec((nblk, 3), lambda i: (i, 0)),
            pl.BlockSpec((nblk, F), lambda i: (i, 0)),
            pl.BlockSpec((H3, F), lambda i: (0, 0)),
            pl.BlockSpec((3, F), lambda i: (0, 0)),
        ],
        pl.BlockSpec((nblk, F), lambda i: (i, 0)),
        (nrows // nblk,),
    )(V, D, x, Wcat, Bstk)


def _cry_fwd(x, w, W1, b1, Wg2, bg2, pows):
    def body(x_ref, w_ref, W1_ref, b1_ref, Wg2_ref, bg2_ref, p_ref,
             g8_ref, h_ref):
        z = jnp.dot(x_ref[...], W1_ref[...], preferred_element_type=jnp.float32)
        z = z + b1_ref[...]
        hid = jnp.where(z > 0, z, 0.01 * z)
        g = jnp.dot(hid, Wg2_ref[...], preferred_element_type=jnp.float32)
        g = g + bg2_ref[...]
        lw = jnp.log(w_ref[...]) * p_ref[...]
        g8_ref[...] = jnp.concatenate(
            [g, lw, jnp.zeros((NBLK, 2), jnp.float32)], axis=1)
        h_ref[...] = hid[:, H3:]

    return _pcall(
        body,
        [jax.ShapeDtypeStruct((N_NODES, 8), jnp.float32),
         jax.ShapeDtypeStruct((N_NODES, H3), jnp.float32)],
        [
            pl.BlockSpec((NBLK, F), lambda i: (i, 0)),
            pl.BlockSpec((NBLK, 1), lambda i: (i, 0)),
            pl.BlockSpec((F, 2 * H3), lambda i: (0, 0)),
            pl.BlockSpec((1, 2 * H3), lambda i: (0, 0)),
            pl.BlockSpec((2 * H3, 3), lambda i: (0, 0)),
            pl.BlockSpec((1, 3), lambda i: (0, 0)),
            pl.BlockSpec((1, 3), lambda i: (0, 0)),
        ],
        [pl.BlockSpec((NBLK, 8), lambda i: (i, 0)),
         pl.BlockSpec((NBLK, H3), lambda i: (i, 0))],
        (N_NODES // NBLK,),
    )(x, w, W1, b1, Wg2, bg2, pows)


def _fuse_wap_heads(heads, din):
    Wg1 = [h["gate"]["hidden"][0]["W"] for h in heads]
    Wm1 = [h["msg"]["hidden"][0]["W"] for h in heads]
    bg1 = [h["gate"]["hidden"][0]["b"] for h in heads]
    bm1 = [h["msg"]["hidden"][0]["b"] for h in heads]
    W1 = jnp.concatenate(Wg1 + Wm1, axis=1)
    b1 = jnp.concatenate(bg1 + bm1)[None, :]
    Wg2 = jnp.zeros((2 * H3, 3), jnp.float32)
    for h in range(3):
        Wg2 = Wg2.at[64 * h:64 * (h + 1), h].set(heads[h]["gate"]["out"]["W"][:, 0])
    bg2 = jnp.stack([h["gate"]["out"]["b"][0] for h in heads])[None, :]
    Wcat = jnp.concatenate([h["msg"]["out"]["W"] for h in heads], axis=0) / 3.0
    Bstk = jnp.stack([h["msg"]["out"]["b"] for h in heads], axis=0) / 3.0
    pows = jnp.stack([h["pow"][0] for h in heads])[None, :]
    return dict(W1=W1, b1=b1, Wg2=Wg2, bg2=bg2, Wcat=Wcat, Bstk=Bstk,
                pows=pows)


def _seg_softmax_agg(G8, Hfea, seg, nseg):
    g = G8[:, :3]
    lw = G8[:, 3:6]
    mg = jax.ops.segment_max(g, seg, num_segments=nseg)
    mg = jnp.maximum(mg, -3.0e38)
    a = jnp.exp(g + lw - mg[seg])
    D = jax.ops.segment_sum(a, seg, num_segments=nseg)
    Vh = [jax.ops.segment_sum(a[:, h:h + 1] * Hfea[:, 64 * h:64 * (h + 1)],
                              seg, num_segments=nseg) for h in range(3)]
    V = jnp.concatenate(Vh, axis=1)
    return V, D


def kernel(elem_weights, elem_fea, self_fea_idx, nbr_fea_idx, cry_elem_idx,
           params):
    w = elem_weights.astype(jnp.float32)
    self_idx = self_fea_idx.astype(jnp.int32)
    nbr_idx = nbr_fea_idx.astype(jnp.int32)
    cry_idx = cry_elem_idx.astype(jnp.int32)

    We = params["embedding"]["W"]
    be = params["embedding"]["b"]
    We64 = jnp.concatenate([We, jnp.zeros((128, 1), jnp.float32)], axis=1)
    be64 = jnp.concatenate([be, jnp.zeros((1,), jnp.float32)])[None, :]
    lastcol = jnp.zeros((1, F), jnp.float32).at[0, F - 1].set(1.0)

    layers = [_fuse_wap_heads(hs, 2 * F) for hs in params["graphs"]]
    cryp = _fuse_wap_heads(params["cry_pool"], F)

    x = _embed(elem_fea, w, We64, be64, lastcol)

    logw = jnp.log(w)

    for lp in layers:
        lwp = logw * lp["pows"]
        xs = x[self_idx]
        xn = x[nbr_idx]
        lwn = lwp[nbr_idx]
        G8, Hfea = _edge_fwd(xs, xn, lwn,
                             lp["W1"][:F], lp["W1"][F:], lp["b1"],
                             lp["Wg2"], lp["bg2"])
        V, D = _seg_softmax_agg(G8, Hfea, self_idx, N_NODES)
        x = _node_fin(V, D, x, lp["Wcat"], lp["Bstk"], N_NODES, NBLK,
                      residual=True)

    G8c, Hc = _cry_fwd(x, w, cryp["W1"], cryp["b1"], cryp["Wg2"],
                       cryp["bg2"], cryp["pows"])
    Vc, Dc = _seg_softmax_agg(G8c, Hc, cry_idx, N_CRYSTALS)
    out = _node_fin(Vc, Dc, jnp.zeros((N_CRYSTALS, F), jnp.float32),
                    cryp["Wcat"], cryp["Bstk"], N_CRYSTALS, 200,
                    residual=False)
    return out


# speedup vs baseline: 2.7699x; 1.0969x over previous
import functools

import jax
import jax.numpy as jnp
import numpy as np
from jax import lax
from jax.experimental import pallas as pl
from jax.experimental.pallas import tpu as pltpu
from jax.experimental.pallas import tpu_sc as plsc

N_NODES = 10000
N_EDGES = 320000
N_CRYSTALS = 2000
F = 64
H3 = 192
NBLK = 1000
EBLK = 3200

_INTERPRET = False


def _pcall(body, out_shape, in_specs, out_specs, grid):
    return pl.pallas_call(
        body,
        out_shape=out_shape,
        in_specs=in_specs,
        out_specs=out_specs,
        grid=grid,
        interpret=_INTERPRET,
    )


def _embed(elem_fea, w, We64, be64, lastcol):
    def body(ef_ref, w_ref, We_ref, be_ref, lc_ref, x_ref):
        x = jnp.dot(ef_ref[...], We_ref[...], preferred_element_type=jnp.float32)
        x = x + be_ref[...]
        x_ref[...] = x + jnp.dot(w_ref[...], lc_ref[...],
                                 preferred_element_type=jnp.float32)

    return _pcall(
        body,
        jax.ShapeDtypeStruct((N_NODES, F), jnp.float32),
        [
            pl.BlockSpec((NBLK, 128), lambda i: (i, 0)),
            pl.BlockSpec((NBLK, 1), lambda i: (i, 0)),
            pl.BlockSpec((128, F), lambda i: (0, 0)),
            pl.BlockSpec((1, F), lambda i: (0, 0)),
            pl.BlockSpec((1, F), lambda i: (0, 0)),
        ],
        pl.BlockSpec((NBLK, F), lambda i: (i, 0)),
        (N_NODES // NBLK,),
    )(elem_fea, w, We64, be64, lastcol)


def _edge_fwd(xs, xn, lwn, W1s, W1n, b1, Wg2, bg2):
    def body(xs_ref, xn_ref, lwn_ref, W1s_ref, W1n_ref, b1_ref, Wg2_ref,
             bg2_ref, g8_ref, h0_ref, h1_ref, h2_ref):
        z = jnp.dot(xs_ref[...], W1s_ref[...], preferred_element_type=jnp.float32)
        z = z + jnp.dot(xn_ref[...], W1n_ref[...], preferred_element_type=jnp.float32)
        z = z + b1_ref[...]
        hid = jnp.where(z > 0, z, 0.01 * z)
        g = jnp.dot(hid, Wg2_ref[...], preferred_element_type=jnp.float32)
        g = g + bg2_ref[...]
        g8_ref[...] = jnp.concatenate(
            [g, lwn_ref[...], jnp.zeros((EBLK, 2), jnp.float32)], axis=1)
        h0_ref[...] = hid[:, H3:H3 + 64]
        h1_ref[...] = hid[:, H3 + 64:H3 + 128]
        h2_ref[...] = hid[:, H3 + 128:]

    return _pcall(
        body,
        [jax.ShapeDtypeStruct((N_EDGES, 8), jnp.float32)] +
        [jax.ShapeDtypeStruct((N_EDGES, F), jnp.float32)] * 3,
        [
            pl.BlockSpec((EBLK, F), lambda i: (i, 0)),
            pl.BlockSpec((EBLK, F), lambda i: (i, 0)),
            pl.BlockSpec((EBLK, 3), lambda i: (i, 0)),
            pl.BlockSpec((F, 2 * H3), lambda i: (0, 0)),
            pl.BlockSpec((F, 2 * H3), lambda i: (0, 0)),
            pl.BlockSpec((1, 2 * H3), lambda i: (0, 0)),
            pl.BlockSpec((2 * H3, 3), lambda i: (0, 0)),
            pl.BlockSpec((1, 3), lambda i: (0, 0)),
        ],
        [pl.BlockSpec((EBLK, 8), lambda i: (i, 0))] +
        [pl.BlockSpec((EBLK, F), lambda i: (i, 0))] * 3,
        (N_EDGES // EBLK,),
    )(xs, xn, lwn, W1s, W1n, b1, Wg2, bg2)


def _node_fin(V, D, x, Wcat, Bstk, nrows, nblk, residual):
    def body2(V_ref, D_ref, x_ref, Wc_ref, Bs_ref, o_ref):
        D = D_ref[...]
        sig = D / (D + 1e-10)
        Vn = V_ref[...] * jnp.repeat(1.0 / (D + 1e-10), F, axis=1)
        o = jnp.dot(Vn, Wc_ref[...], preferred_element_type=jnp.float32)
        o = o + jnp.dot(sig, Bs_ref[...], preferred_element_type=jnp.float32)
        if residual:
            o = o + x_ref[...]
        o_ref[...] = o

    return _pcall(
        body2,
        jax.ShapeDtypeStruct((nrows, F), jnp.float32),
        [
            pl.BlockSpec((nblk, H3), lambda i: (i, 0)),
            pl.BlockSpec((nblk, 3), lambda i: (i, 0)),
            pl.BlockSpec((nblk, F), lambda i: (i, 0)),
            pl.BlockSpec((H3, F), lambda i: (0, 0)),
            pl.BlockSpec((3, F), lambda i: (0, 0)),
        ],
        pl.BlockSpec((nblk, F), lambda i: (i, 0)),
        (nrows // nblk,),
    )(V, D, x, Wcat, Bstk)


def _cry_fwd(x, w, W1, b1, Wg2, bg2, pows):
    def body(x_ref, w_ref, W1_ref, b1_ref, Wg2_ref, bg2_ref, p_ref,
             g8_ref, h0_ref, h1_ref, h2_ref):
        z = jnp.dot(x_ref[...], W1_ref[...], preferred_element_type=jnp.float32)
        z = z + b1_ref[...]
        hid = jnp.where(z > 0, z, 0.01 * z)
        g = jnp.dot(hid, Wg2_ref[...], preferred_element_type=jnp.float32)
        g = g + bg2_ref[...]
        lw = jnp.log(w_ref[...]) * p_ref[...]
        g8_ref[...] = jnp.concatenate(
            [g, lw, jnp.zeros((NBLK, 2), jnp.float32)], axis=1)
        h0_ref[...] = hid[:, H3:H3 + 64]
        h1_ref[...] = hid[:, H3 + 64:H3 + 128]
        h2_ref[...] = hid[:, H3 + 128:]

    return _pcall(
        body,
        [jax.ShapeDtypeStruct((N_NODES, 8), jnp.float32)] +
        [jax.ShapeDtypeStruct((N_NODES, F), jnp.float32)] * 3,
        [
            pl.BlockSpec((NBLK, F), lambda i: (i, 0)),
            pl.BlockSpec((NBLK, 1), lambda i: (i, 0)),
            pl.BlockSpec((F, 2 * H3), lambda i: (0, 0)),
            pl.BlockSpec((1, 2 * H3), lambda i: (0, 0)),
            pl.BlockSpec((2 * H3, 3), lambda i: (0, 0)),
            pl.BlockSpec((1, 3), lambda i: (0, 0)),
            pl.BlockSpec((1, 3), lambda i: (0, 0)),
        ],
        [pl.BlockSpec((NBLK, 8), lambda i: (i, 0))] +
        [pl.BlockSpec((NBLK, F), lambda i: (i, 0))] * 3,
        (N_NODES // NBLK,),
    )(x, w, W1, b1, Wg2, bg2, pows)


def _fuse_wap_heads(heads, din):
    Wg1 = [h["gate"]["hidden"][0]["W"] for h in heads]
    Wm1 = [h["msg"]["hidden"][0]["W"] for h in heads]
    bg1 = [h["gate"]["hidden"][0]["b"] for h in heads]
    bm1 = [h["msg"]["hidden"][0]["b"] for h in heads]
    W1 = jnp.concatenate(Wg1 + Wm1, axis=1)
    b1 = jnp.concatenate(bg1 + bm1)[None, :]
    Wg2 = jnp.zeros((2 * H3, 3), jnp.float32)
    for h in range(3):
        Wg2 = Wg2.at[64 * h:64 * (h + 1), h].set(heads[h]["gate"]["out"]["W"][:, 0])
    bg2 = jnp.stack([h["gate"]["out"]["b"][0] for h in heads])[None, :]
    Wcat = jnp.concatenate([h["msg"]["out"]["W"] for h in heads], axis=0) / 3.0
    Bstk = jnp.stack([h["msg"]["out"]["b"] for h in heads], axis=0) / 3.0
    pows = jnp.stack([h["pow"][0] for h in heads])[None, :]
    return dict(W1=W1, b1=b1, Wg2=Wg2, bg2=bg2, Wcat=Wcat, Bstk=Bstk,
                pows=pows)


def _seg_softmax_agg(G8, Hs, seg, nseg):
    g = G8[:, :3]
    lw = G8[:, 3:6]
    mg = jax.ops.segment_max(g, seg, num_segments=nseg,
                             indices_are_sorted=True)
    mg = jnp.maximum(mg, -3.0e38)
    a = jnp.exp(g + lw - mg.at[seg].get(mode="promise_in_bounds",
                                        indices_are_sorted=True))
    D = jax.ops.segment_sum(a, seg, num_segments=nseg,
                            indices_are_sorted=True)
    Vh = [jax.ops.segment_sum(a[:, h:h + 1] * Hs[h], seg, num_segments=nseg,
                              indices_are_sorted=True) for h in range(3)]
    return jnp.concatenate(Vh, axis=1), D


def kernel(elem_weights, elem_fea, self_fea_idx, nbr_fea_idx, cry_elem_idx,
           params):
    w = elem_weights.astype(jnp.float32)
    self_idx = self_fea_idx.astype(jnp.int32)
    nbr_idx = nbr_fea_idx.astype(jnp.int32)
    cry_idx = cry_elem_idx.astype(jnp.int32)

    We = params["embedding"]["W"]
    be = params["embedding"]["b"]
    We64 = jnp.concatenate([We, jnp.zeros((128, 1), jnp.float32)], axis=1)
    be64 = jnp.concatenate([be, jnp.zeros((1,), jnp.float32)])[None, :]
    lastcol = jnp.zeros((1, F), jnp.float32).at[0, F - 1].set(1.0)

    layers = [_fuse_wap_heads(hs, 2 * F) for hs in params["graphs"]]
    cryp = _fuse_wap_heads(params["cry_pool"], F)

    x = _embed(elem_fea, w, We64, be64, lastcol)

    logw = jnp.log(w)

    for lp in layers:
        lwp = logw * lp["pows"]
        xs = x.at[self_idx].get(mode="promise_in_bounds",
                                indices_are_sorted=True)
        xn = x.at[nbr_idx].get(mode="promise_in_bounds")
        lwn = lwp.at[nbr_idx].get(mode="promise_in_bounds")
        G8, H0, H1, H2 = _edge_fwd(xs, xn, lwn,
                                   lp["W1"][:F], lp["W1"][F:], lp["b1"],
                                   lp["Wg2"], lp["bg2"])
        V, D = _seg_softmax_agg(G8, (H0, H1, H2), self_idx, N_NODES)
        x = _node_fin(V, D, x, lp["Wcat"], lp["Bstk"], N_NODES, NBLK,
                      residual=True)

    G8c, Hc0, Hc1, Hc2 = _cry_fwd(x, w, cryp["W1"], cryp["b1"], cryp["Wg2"],
                                  cryp["bg2"], cryp["pows"])
    Vc, Dc = _seg_softmax_agg(G8c, (Hc0, Hc1, Hc2), cry_idx, N_CRYSTALS)
    out = _node_fin(Vc, Dc, jnp.zeros((N_CRYSTALS, F), jnp.float32),
                    cryp["Wcat"], cryp["Bstk"], N_CRYSTALS, 200,
                    residual=False)
    return out


# speedup vs baseline: 3.1146x; 1.1244x over previous
import functools

import jax
import jax.numpy as jnp
import numpy as np
from jax import lax
from jax.experimental import pallas as pl
from jax.experimental.pallas import tpu as pltpu
from jax.experimental.pallas import tpu_sc as plsc

N_NODES = 10000
N_EDGES = 320000
N_CRYSTALS = 2000
F = 64
H3 = 192
NBLK = 1000
EBLK = 3200

_INTERPRET = False


def _pcall(body, out_shape, in_specs, out_specs, grid):
    return pl.pallas_call(
        body,
        out_shape=out_shape,
        in_specs=in_specs,
        out_specs=out_specs,
        grid=grid,
        interpret=_INTERPRET,
    )


def _embed(elem_fea, w, We64, be64, lastcol):
    def body(ef_ref, w_ref, We_ref, be_ref, lc_ref, x_ref):
        x = jnp.dot(ef_ref[...], We_ref[...], preferred_element_type=jnp.float32)
        x = x + be_ref[...]
        x_ref[...] = x + jnp.dot(w_ref[...], lc_ref[...],
                                 preferred_element_type=jnp.float32)

    return _pcall(
        body,
        jax.ShapeDtypeStruct((N_NODES, F), jnp.float32),
        [
            pl.BlockSpec((NBLK, 128), lambda i: (i, 0)),
            pl.BlockSpec((NBLK, 1), lambda i: (i, 0)),
            pl.BlockSpec((128, F), lambda i: (0, 0)),
            pl.BlockSpec((1, F), lambda i: (0, 0)),
            pl.BlockSpec((1, F), lambda i: (0, 0)),
        ],
        pl.BlockSpec((NBLK, F), lambda i: (i, 0)),
        (N_NODES // NBLK,),
    )(elem_fea, w, We64, be64, lastcol)


def _edge_fwd(xs, xn, lwn, W1s, W1n, b1, Wg2, bg2):
    def body(xs_ref, xn_ref, lwn_ref, W1s_ref, W1n_ref, b1_ref, Wg2_ref,
             bg2_ref, g8_ref, h0_ref, h1_ref, h2_ref):
        z = jnp.dot(xs_ref[...], W1s_ref[...], preferred_element_type=jnp.float32)
        z = z + jnp.dot(xn_ref[...], W1n_ref[...], preferred_element_type=jnp.float32)
        z = z + b1_ref[...]
        hid = jnp.where(z > 0, z, 0.01 * z)
        g = jnp.dot(hid, Wg2_ref[...], preferred_element_type=jnp.float32)
        g = g + bg2_ref[...]
        g8_ref[...] = jnp.concatenate(
            [g, lwn_ref[...], jnp.zeros((EBLK, 2), jnp.float32)], axis=1)
        h0_ref[...] = hid[:, H3:H3 + 64]
        h1_ref[...] = hid[:, H3 + 64:H3 + 128]
        h2_ref[...] = hid[:, H3 + 128:]

    return _pcall(
        body,
        [jax.ShapeDtypeStruct((N_EDGES, 8), jnp.float32)] +
        [jax.ShapeDtypeStruct((N_EDGES, F), jnp.float32)] * 3,
        [
            pl.BlockSpec((EBLK, F), lambda i: (i, 0)),
            pl.BlockSpec((EBLK, F), lambda i: (i, 0)),
            pl.BlockSpec((EBLK, 3), lambda i: (i, 0)),
            pl.BlockSpec((F, 2 * H3), lambda i: (0, 0)),
            pl.BlockSpec((F, 2 * H3), lambda i: (0, 0)),
            pl.BlockSpec((1, 2 * H3), lambda i: (0, 0)),
            pl.BlockSpec((2 * H3, 3), lambda i: (0, 0)),
            pl.BlockSpec((1, 3), lambda i: (0, 0)),
        ],
        [pl.BlockSpec((EBLK, 8), lambda i: (i, 0))] +
        [pl.BlockSpec((EBLK, F), lambda i: (i, 0))] * 3,
        (N_EDGES // EBLK,),
    )(xs, xn, lwn, W1s, W1n, b1, Wg2, bg2)


def _node_fin(V, D, x, Wcat, Bstk, nrows, nblk, residual):
    def body2(V_ref, D_ref, x_ref, Wc_ref, Bs_ref, o_ref):
        D = D_ref[...]
        sig = D / (D + 1e-10)
        Vn = V_ref[...] * jnp.repeat(1.0 / (D + 1e-10), F, axis=1)
        o = jnp.dot(Vn, Wc_ref[...], preferred_element_type=jnp.float32)
        o = o + jnp.dot(sig, Bs_ref[...], preferred_element_type=jnp.float32)
        if residual:
            o = o + x_ref[...]
        o_ref[...] = o

    return _pcall(
        body2,
        jax.ShapeDtypeStruct((nrows, F), jnp.float32),
        [
            pl.BlockSpec((nblk, H3), lambda i: (i, 0)),
            pl.BlockSpec((nblk, 3), lambda i: (i, 0)),
            pl.BlockSpec((nblk, F), lambda i: (i, 0)),
            pl.BlockSpec((H3, F), lambda i: (0, 0)),
            pl.BlockSpec((3, F), lambda i: (0, 0)),
        ],
        pl.BlockSpec((nblk, F), lambda i: (i, 0)),
        (nrows // nblk,),
    )(V, D, x, Wcat, Bstk)


def _cry_fwd(x, w, W1, b1, Wg2, bg2, pows):
    def body(x_ref, w_ref, W1_ref, b1_ref, Wg2_ref, bg2_ref, p_ref,
             g8_ref, h0_ref, h1_ref, h2_ref):
        z = jnp.dot(x_ref[...], W1_ref[...], preferred_element_type=jnp.float32)
        z = z + b1_ref[...]
        hid = jnp.where(z > 0, z, 0.01 * z)
        g = jnp.dot(hid, Wg2_ref[...], preferred_element_type=jnp.float32)
        g = g + bg2_ref[...]
        lw = jnp.log(w_ref[...]) * p_ref[...]
        g8_ref[...] = jnp.concatenate(
            [g, lw, jnp.zeros((NBLK, 2), jnp.float32)], axis=1)
        h0_ref[...] = hid[:, H3:H3 + 64]
        h1_ref[...] = hid[:, H3 + 64:H3 + 128]
        h2_ref[...] = hid[:, H3 + 128:]

    return _pcall(
        body,
        [jax.ShapeDtypeStruct((N_NODES, 8), jnp.float32)] +
        [jax.ShapeDtypeStruct((N_NODES, F), jnp.float32)] * 3,
        [
            pl.BlockSpec((NBLK, F), lambda i: (i, 0)),
            pl.BlockSpec((NBLK, 1), lambda i: (i, 0)),
            pl.BlockSpec((F, 2 * H3), lambda i: (0, 0)),
            pl.BlockSpec((1, 2 * H3), lambda i: (0, 0)),
            pl.BlockSpec((2 * H3, 3), lambda i: (0, 0)),
            pl.BlockSpec((1, 3), lambda i: (0, 0)),
            pl.BlockSpec((1, 3), lambda i: (0, 0)),
        ],
        [pl.BlockSpec((NBLK, 8), lambda i: (i, 0))] +
        [pl.BlockSpec((NBLK, F), lambda i: (i, 0))] * 3,
        (N_NODES // NBLK,),
    )(x, w, W1, b1, Wg2, bg2, pows)


def _fuse_wap_heads(heads, din):
    Wg1 = [h["gate"]["hidden"][0]["W"] for h in heads]
    Wm1 = [h["msg"]["hidden"][0]["W"] for h in heads]
    bg1 = [h["gate"]["hidden"][0]["b"] for h in heads]
    bm1 = [h["msg"]["hidden"][0]["b"] for h in heads]
    W1 = jnp.concatenate(Wg1 + Wm1, axis=1)
    b1 = jnp.concatenate(bg1 + bm1)[None, :]
    Wg2 = jnp.zeros((2 * H3, 3), jnp.float32)
    for h in range(3):
        Wg2 = Wg2.at[64 * h:64 * (h + 1), h].set(heads[h]["gate"]["out"]["W"][:, 0])
    bg2 = jnp.stack([h["gate"]["out"]["b"][0] for h in heads])[None, :]
    Wcat = jnp.concatenate([h["msg"]["out"]["W"] for h in heads], axis=0) / 3.0
    Bstk = jnp.stack([h["msg"]["out"]["b"] for h in heads], axis=0) / 3.0
    pows = jnp.stack([h["pow"][0] for h in heads])[None, :]
    return dict(W1=W1, b1=b1, Wg2=Wg2, bg2=bg2, Wcat=Wcat, Bstk=Bstk,
                pows=pows)


def _seg_softmax_agg(G8, Hs, seg, nseg):
    g = G8[:, :3]
    lw = G8[:, 3:6]
    mg = jax.ops.segment_max(g, seg, num_segments=nseg,
                             indices_are_sorted=True)
    mg = jnp.maximum(mg, -3.0e38)
    a = jnp.exp(g + lw - mg.at[seg].get(mode="promise_in_bounds",
                                        indices_are_sorted=True))
    AH = jnp.concatenate([a[:, h:h + 1] * Hs[h] for h in range(3)] + [a],
                         axis=1)
    S = jax.ops.segment_sum(AH, seg, num_segments=nseg,
                            indices_are_sorted=True)
    return S[:, :H3], S[:, H3:]


def kernel(elem_weights, elem_fea, self_fea_idx, nbr_fea_idx, cry_elem_idx,
           params):
    w = elem_weights.astype(jnp.float32)
    self_idx = self_fea_idx.astype(jnp.int32)
    nbr_idx = nbr_fea_idx.astype(jnp.int32)
    cry_idx = cry_elem_idx.astype(jnp.int32)

    We = params["embedding"]["W"]
    be = params["embedding"]["b"]
    We64 = jnp.concatenate([We, jnp.zeros((128, 1), jnp.float32)], axis=1)
    be64 = jnp.concatenate([be, jnp.zeros((1,), jnp.float32)])[None, :]
    lastcol = jnp.zeros((1, F), jnp.float32).at[0, F - 1].set(1.0)

    layers = [_fuse_wap_heads(hs, 2 * F) for hs in params["graphs"]]
    cryp = _fuse_wap_heads(params["cry_pool"], F)

    x = _embed(elem_fea, w, We64, be64, lastcol)

    logw = jnp.log(w)

    for lp in layers:
        lwp = logw * lp["pows"]
        xaug = jnp.concatenate([x, lwp], axis=1)
        xs = x.at[self_idx].get(mode="promise_in_bounds",
                                indices_are_sorted=True)
        xna = xaug.at[nbr_idx].get(mode="promise_in_bounds")
        xn = xna[:, :F]
        lwn = xna[:, F:]
        G8, H0, H1, H2 = _edge_fwd(xs, xn, lwn,
                                   lp["W1"][:F], lp["W1"][F:], lp["b1"],
                                   lp["Wg2"], lp["bg2"])
        V, D = _seg_softmax_agg(G8, (H0, H1, H2), self_idx, N_NODES)
        x = _node_fin(V, D, x, lp["Wcat"], lp["Bstk"], N_NODES, NBLK,
                      residual=True)

    G8c, Hc0, Hc1, Hc2 = _cry_fwd(x, w, cryp["W1"], cryp["b1"], cryp["Wg2"],
                                  cryp["bg2"], cryp["pows"])
    Vc, Dc = _seg_softmax_agg(G8c, (Hc0, Hc1, Hc2), cry_idx, N_CRYSTALS)
    out = _node_fin(Vc, Dc, jnp.zeros((N_CRYSTALS, F), jnp.float32),
                    cryp["Wcat"], cryp["Bstk"], N_CRYSTALS, 200,
                    residual=False)
    return out
